# baseline (speedup 1.0000x reference)
import functools

import jax
import jax.numpy as jnp
from jax import lax
from jax.experimental import pallas as pl
from jax.experimental.pallas import tpu as pltpu
from jax.experimental.pallas import tpu_sc as plsc

EMB_DIM = 64
RANK = 16
SCALING = 32.0 / 16.0
NUM_CORES = 2
NUM_SUBCORES = 16
NW = NUM_CORES * NUM_SUBCORES
CHUNK = 128


def _make_lora_embed(n_idx):
  assert n_idx % (NW * 2 * CHUNK) == 0
  per_w = n_idx // NW
  n_step = per_w // CHUNK
  mesh = plsc.VectorSubcoreMesh(core_axis_name="c", subcore_axis_name="s")

  @functools.partial(
      pl.kernel,
      out_type=jax.ShapeDtypeStruct((n_idx, EMB_DIM), jnp.float32),
      mesh=mesh,
      scratch_types=[
          pltpu.VMEM((n_step, CHUNK), jnp.int32),
          pltpu.VMEM((2, CHUNK, EMB_DIM), jnp.float32),
          pltpu.VMEM((2, CHUNK, RANK), jnp.float32),
          pltpu.VMEM((RANK, EMB_DIM), jnp.float32),
          pltpu.SemaphoreType.DMA,
          pltpu.SemaphoreType.DMA,
          pltpu.SemaphoreType.DMA,
          pltpu.SemaphoreType.DMA,
          pltpu.SemaphoreType.DMA,
          pltpu.SemaphoreType.DMA,
      ],
      compiler_params=pltpu.CompilerParams(use_tc_tiling_on_sc=False),
  )
  def lora_embed(x_hbm, w_hbm, a_hbm, b_hbm, out_hbm,
                 idx_all, rows_v, lora_v, bmat_v,
                 gw0, gw1, ga0, ga1, ws0, ws1):
    gw = (gw0, gw1)
    ga = (ga0, ga1)
    ws = (ws0, ws1)
    wid = lax.axis_index("s") * NUM_CORES + lax.axis_index("c")
    base = wid * per_w
    pltpu.sync_copy(b_hbm, bmat_v)
    pltpu.sync_copy(x_hbm.at[wid], idx_all)

    def issue(i, s):
      pltpu.async_copy(w_hbm.at[idx_all.at[i]], rows_v.at[s], gw[s])
      pltpu.async_copy(a_hbm.at[idx_all.at[i]], lora_v.at[s], ga[s])

    def drain_gathers(s):
      pltpu.make_async_copy(w_hbm.at[pl.ds(0, CHUNK)], rows_v.at[s], gw[s]).wait()
      pltpu.make_async_copy(a_hbm.at[pl.ds(0, CHUNK)], lora_v.at[s], ga[s]).wait()

    def drain_write(s):
      pltpu.make_async_copy(
          rows_v.at[s], out_hbm.at[pl.ds(0, CHUNK)], ws[s]).wait()

    bregs = [
        tuple(bmat_v[k, pl.ds(16 * j, 16)] for k in range(8 * h, 8 * h + 8)
              for j in range(4))
        for h in range(2)
    ]

    def make_pass(s, h):
      def row(c, bcarry):
        lr = lora_v[s, c, pl.ds(0, RANK)]
        acc = [rows_v[s, c, pl.ds(16 * j, 16)] for j in range(4)]
        acc2 = [None] * 4
        for kk in range(8):
          k = 8 * h + kk
          sk = lr[jnp.full((16,), k, jnp.int32)]
          for j in range(4):
            t = sk * bcarry[4 * kk + j]
            if kk == 0:
              acc[j] = acc[j] + t
            elif kk == 1:
              acc2[j] = t
            elif kk % 2 == 0:
              acc[j] = acc[j] + t
            else:
              acc2[j] = acc2[j] + t
        for j in range(4):
          rows_v[s, c, pl.ds(16 * j, 16)] = acc[j] + acc2[j]
        return bcarry

      return row

    def compute(s):
      for h in range(2):
        lax.fori_loop(0, CHUNK, make_pass(s, h), bregs[h], unroll=False)

    issue(0, 0)

    def outer(g, carry):
      for b in range(2):
        i = 2 * g + b
        nxt = i + 1

        @pl.when(nxt < n_step)
        def _():
          if b == 1:
            drain_write(0)
          else:

            @pl.when(i >= 1)
            def _():
              drain_write(1)

          issue(nxt, 1 - b)

        drain_gathers(b)
        compute(b)
        off = base + i * CHUNK
        pltpu.async_copy(rows_v.at[b], out_hbm.at[pl.ds(off, CHUNK)], ws[b])
      return carry

    lax.fori_loop(0, n_step // 2, outer, 0, unroll=False)
    drain_write(0)
    drain_write(1)

  return lora_embed


def kernel(x, weight, lora_A, lora_B):
  b, l = x.shape
  n = b * l
  xf = x.reshape(NW, n // (NW * CHUNK), CHUNK).astype(jnp.int32)
  b_scaled = (SCALING * lora_B).astype(jnp.float32)
  out = _make_lora_embed(n)(xf, weight, lora_A, b_scaled)
  return out.reshape(b, l, EMB_DIM)


# speedup vs baseline: 1.1914x; 1.1914x over previous
import functools

import jax
import jax.numpy as jnp
from jax import lax
from jax.experimental import pallas as pl
from jax.experimental.pallas import tpu as pltpu
from jax.experimental.pallas import tpu_sc as plsc

NUM_EMB = 1000000
EMB_DIM = 64
RANK = 16
SCALING = 32.0 / 16.0
NUM_CORES = 2
NUM_SUBCORES = 16
NW = NUM_CORES * NUM_SUBCORES
CHUNK = 128
FUSE_BM = 8000


def _fuse_block(w_ref, a_ref, b_ref, f_ref):
  f_ref[...] = w_ref[...] + SCALING * jnp.dot(
      a_ref[...], b_ref[...], preferred_element_type=jnp.float32,
      precision=lax.Precision.HIGHEST)


def _build_fused_table(weight, lora_A, lora_B):
  n = weight.shape[0]
  assert n % FUSE_BM == 0
  grid = n // FUSE_BM
  return pl.pallas_call(
      _fuse_block,
      grid=(grid,),
      in_specs=[
          pl.BlockSpec((FUSE_BM, EMB_DIM), lambda i: (i, 0)),
          pl.BlockSpec((FUSE_BM, RANK), lambda i: (i, 0)),
          pl.BlockSpec((RANK, EMB_DIM), lambda i: (0, 0)),
      ],
      out_specs=pl.BlockSpec((FUSE_BM, EMB_DIM), lambda i: (i, 0)),
      out_shape=jax.ShapeDtypeStruct((n, EMB_DIM), jnp.float32),
  )(weight, lora_A, lora_B)


def _make_gather(n_idx):
  assert n_idx % (NW * 2 * CHUNK) == 0
  per_w = n_idx // NW
  n_step = per_w // CHUNK
  mesh = plsc.VectorSubcoreMesh(core_axis_name="c", subcore_axis_name="s")

  @functools.partial(
      pl.kernel,
      out_type=jax.ShapeDtypeStruct((n_idx, EMB_DIM), jnp.float32),
      mesh=mesh,
      scratch_types=[
          pltpu.VMEM((n_step, CHUNK), jnp.int32),
          pltpu.VMEM((2, CHUNK, EMB_DIM), jnp.float32),
          pltpu.SemaphoreType.DMA,
          pltpu.SemaphoreType.DMA,
          pltpu.SemaphoreType.DMA,
          pltpu.SemaphoreType.DMA,
      ],
      compiler_params=pltpu.CompilerParams(use_tc_tiling_on_sc=False),
  )
  def gather_rows(x_hbm, f_hbm, out_hbm, idx_all, rows_v, gs0, gs1, ws0, ws1):
    gs = (gs0, gs1)
    ws = (ws0, ws1)
    wid = lax.axis_index("s") * NUM_CORES + lax.axis_index("c")
    base = wid * per_w
    pltpu.sync_copy(x_hbm.at[wid], idx_all)

    def issue(i, s):
      pltpu.async_copy(f_hbm.at[idx_all.at[i]], rows_v.at[s], gs[s])

    def drain_gather(s):
      pltpu.make_async_copy(f_hbm.at[pl.ds(0, CHUNK)], rows_v.at[s], gs[s]).wait()

    def drain_write(s):
      pltpu.make_async_copy(
          rows_v.at[s], out_hbm.at[pl.ds(0, CHUNK)], ws[s]).wait()

    issue(0, 0)

    def outer(g, carry):
      for b in range(2):
        i = 2 * g + b
        nxt = i + 1

        @pl.when(nxt < n_step)
        def _():
          if b == 1:
            drain_write(0)
          else:

            @pl.when(i >= 1)
            def _():
              drain_write(1)

          issue(nxt, 1 - b)

        drain_gather(b)
        off = base + i * CHUNK
        pltpu.async_copy(rows_v.at[b], out_hbm.at[pl.ds(off, CHUNK)], ws[b])
      return carry

    lax.fori_loop(0, n_step // 2, outer, 0, unroll=False)
    drain_write(0)
    drain_write(1)

  return gather_rows


def kernel(x, weight, lora_A, lora_B):
  b, l = x.shape
  n = b * l
  xf = x.reshape(NW, n // (NW * CHUNK), CHUNK).astype(jnp.int32)
  fused = _build_fused_table(weight, lora_A, lora_B)
  out = _make_gather(n)(xf, fused)
  return out.reshape(b, l, EMB_DIM)


# speedup vs baseline: 1.2256x; 1.0287x over previous
import functools

import jax
import jax.numpy as jnp
from jax import lax
from jax.experimental import pallas as pl
from jax.experimental.pallas import tpu as pltpu
from jax.experimental.pallas import tpu_sc as plsc

NUM_EMB = 1000000
EMB_DIM = 64
RANK = 16
SCALING = 32.0 / 16.0
NUM_CORES = 2
NUM_SUBCORES = 16
NW = NUM_CORES * NUM_SUBCORES
CHUNK = 128
FUSE_BM = 8000


def _fuse_block(w_ref, a_ref, b_ref, f_ref):
  f_ref[...] = w_ref[...] + SCALING * jnp.dot(
      a_ref[...], b_ref[...], preferred_element_type=jnp.float32)


def _build_fused_table(weight, lora_A, lora_B):
  n = weight.shape[0]
  assert n % FUSE_BM == 0
  grid = n // FUSE_BM
  return pl.pallas_call(
      _fuse_block,
      grid=(grid,),
      in_specs=[
          pl.BlockSpec((FUSE_BM, EMB_DIM), lambda i: (i, 0)),
          pl.BlockSpec((FUSE_BM, RANK), lambda i: (i, 0)),
          pl.BlockSpec((RANK, EMB_DIM), lambda i: (0, 0)),
      ],
      out_specs=pl.BlockSpec((FUSE_BM, EMB_DIM), lambda i: (i, 0)),
      out_shape=jax.ShapeDtypeStruct((n, EMB_DIM), jnp.float32),
  )(weight, lora_A, lora_B)


def _make_gather(n_idx):
  assert n_idx % (NW * 2 * CHUNK) == 0
  per_w = n_idx // NW
  n_step = per_w // CHUNK
  mesh = plsc.VectorSubcoreMesh(core_axis_name="c", subcore_axis_name="s")

  @functools.partial(
      pl.kernel,
      out_type=jax.ShapeDtypeStruct((n_idx, EMB_DIM), jnp.float32),
      mesh=mesh,
      scratch_types=[
          pltpu.VMEM((n_step, CHUNK), jnp.int32),
          pltpu.VMEM((2, CHUNK, EMB_DIM), jnp.float32),
          pltpu.SemaphoreType.DMA,
          pltpu.SemaphoreType.DMA,
          pltpu.SemaphoreType.DMA,
          pltpu.SemaphoreType.DMA,
      ],
      compiler_params=pltpu.CompilerParams(use_tc_tiling_on_sc=False),
  )
  def gather_rows(x_hbm, f_hbm, out_hbm, idx_all, rows_v, gs0, gs1, ws0, ws1):
    gs = (gs0, gs1)
    ws = (ws0, ws1)
    wid = lax.axis_index("s") * NUM_CORES + lax.axis_index("c")
    base = wid * per_w
    pltpu.sync_copy(x_hbm.at[wid], idx_all)

    def issue(i, s):
      pltpu.async_copy(f_hbm.at[idx_all.at[i]], rows_v.at[s], gs[s])

    def drain_gather(s):
      pltpu.make_async_copy(f_hbm.at[pl.ds(0, CHUNK)], rows_v.at[s], gs[s]).wait()

    def drain_write(s):
      pltpu.make_async_copy(
          rows_v.at[s], out_hbm.at[pl.ds(0, CHUNK)], ws[s]).wait()

    issue(0, 0)

    def outer(g, carry):
      for b in range(2):
        i = 2 * g + b
        nxt = i + 1

        @pl.when(nxt < n_step)
        def _():
          if b == 1:
            drain_write(0)
          else:

            @pl.when(i >= 1)
            def _():
              drain_write(1)

          issue(nxt, 1 - b)

        drain_gather(b)
        off = base + i * CHUNK
        pltpu.async_copy(rows_v.at[b], out_hbm.at[pl.ds(off, CHUNK)], ws[b])
      return carry

    lax.fori_loop(0, n_step // 2, outer, 0, unroll=False)
    drain_write(0)
    drain_write(1)

  return gather_rows


def kernel(x, weight, lora_A, lora_B):
  b, l = x.shape
  n = b * l
  xf = x.reshape(NW, n // (NW * CHUNK), CHUNK).astype(jnp.int32)
  fused = _build_fused_table(weight, lora_A, lora_B)
  out = _make_gather(n)(xf, fused)
  return out.reshape(b, l, EMB_DIM)


# speedup vs baseline: 1.9116x; 1.5597x over previous
import functools

import jax
import jax.numpy as jnp
from jax import lax
from jax.experimental import pallas as pl
from jax.experimental.pallas import tpu as pltpu
from jax.experimental.pallas import tpu_sc as plsc

NUM_EMB = 1000000
EMB_DIM = 64
RANK = 16
SCALING = 32.0 / 16.0
NUM_CORES = 2
NUM_SUBCORES = 16
NW = NUM_CORES * NUM_SUBCORES
CHUNK = 128
FUSE_BM = 8192


def _fuse_block(wt_ref, at_ref, bt_ref, f_ref):
  c = wt_ref[...] + SCALING * jnp.dot(
      bt_ref[...], at_ref[...], preferred_element_type=jnp.float32)
  f_ref[...] = c.T


def _build_fused_table(weight, lora_A, lora_B):
  n = weight.shape[0]
  grid = (n + FUSE_BM - 1) // FUSE_BM
  return pl.pallas_call(
      _fuse_block,
      grid=(grid,),
      in_specs=[
          pl.BlockSpec((EMB_DIM, FUSE_BM), lambda i: (0, i)),
          pl.BlockSpec((RANK, FUSE_BM), lambda i: (0, i)),
          pl.BlockSpec((EMB_DIM, RANK), lambda i: (0, 0)),
      ],
      out_specs=pl.BlockSpec((FUSE_BM, EMB_DIM), lambda i: (i, 0)),
      out_shape=jax.ShapeDtypeStruct((n, EMB_DIM), jnp.float32),
  )(weight.T, lora_A.T, lora_B.T)


def _make_gather(n_idx):
  assert n_idx % (NW * 2 * CHUNK) == 0
  per_w = n_idx // NW
  n_step = per_w // CHUNK
  mesh = plsc.VectorSubcoreMesh(core_axis_name="c", subcore_axis_name="s")

  @functools.partial(
      pl.kernel,
      out_type=jax.ShapeDtypeStruct((n_idx, EMB_DIM), jnp.float32),
      mesh=mesh,
      scratch_types=[
          pltpu.VMEM((n_step, CHUNK), jnp.int32),
          pltpu.VMEM((2, CHUNK, EMB_DIM), jnp.float32),
          pltpu.SemaphoreType.DMA,
          pltpu.SemaphoreType.DMA,
          pltpu.SemaphoreType.DMA,
          pltpu.SemaphoreType.DMA,
      ],
      compiler_params=pltpu.CompilerParams(use_tc_tiling_on_sc=False),
  )
  def gather_rows(x_hbm, f_hbm, out_hbm, idx_all, rows_v, gs0, gs1, ws0, ws1):
    gs = (gs0, gs1)
    ws = (ws0, ws1)
    wid = lax.axis_index("s") * NUM_CORES + lax.axis_index("c")
    base = wid * per_w
    pltpu.sync_copy(x_hbm.at[wid], idx_all)

    def issue(i, s):
      pltpu.async_copy(f_hbm.at[idx_all.at[i]], rows_v.at[s], gs[s])

    def drain_gather(s):
      pltpu.make_async_copy(f_hbm.at[pl.ds(0, CHUNK)], rows_v.at[s], gs[s]).wait()

    def drain_write(s):
      pltpu.make_async_copy(
          rows_v.at[s], out_hbm.at[pl.ds(0, CHUNK)], ws[s]).wait()

    issue(0, 0)

    def outer(g, carry):
      for b in range(2):
        i = 2 * g + b
        nxt = i + 1

        @pl.when(nxt < n_step)
        def _():
          if b == 1:
            drain_write(0)
          else:

            @pl.when(i >= 1)
            def _():
              drain_write(1)

          issue(nxt, 1 - b)

        drain_gather(b)
        off = base + i * CHUNK
        pltpu.async_copy(rows_v.at[b], out_hbm.at[pl.ds(off, CHUNK)], ws[b])
      return carry

    lax.fori_loop(0, n_step // 2, outer, 0, unroll=False)
    drain_write(0)
    drain_write(1)

  return gather_rows


def kernel(x, weight, lora_A, lora_B):
  b, l = x.shape
  n = b * l
  xf = x.reshape(NW, n // (NW * CHUNK), CHUNK).astype(jnp.int32)
  fused = _build_fused_table(weight, lora_A, lora_B)
  out = _make_gather(n)(xf, fused)
  return out.reshape(b, l, EMB_DIM)


# speedup vs baseline: 2.6544x; 1.3886x over previous
import functools

import jax
import jax.numpy as jnp
from jax import lax
from jax.experimental import pallas as pl
from jax.experimental.pallas import tpu as pltpu
from jax.experimental.pallas import tpu_sc as plsc

NUM_EMB = 1000000
EMB_DIM = 64
RANK = 16
SCALING = 32.0 / 16.0
NUM_CORES = 2
NUM_SUBCORES = 16
NW = NUM_CORES * NUM_SUBCORES
CHUNK = 128
PACK_BN = 4096
PACK_C = 499712


def _fuse_block(wtA_ref, wtB_ref, atA_ref, atB_ref, bt_ref, f_ref):
  cA = wtA_ref[...] + SCALING * jnp.dot(
      bt_ref[...], atA_ref[...], preferred_element_type=jnp.float32)
  cB = wtB_ref[...] + SCALING * jnp.dot(
      bt_ref[...], atB_ref[...], preferred_element_type=jnp.float32)
  f_ref[...] = jnp.concatenate([cA.T, cB.T], axis=1)


def _build_fused_table(weight, lora_A, lora_B):
  n = weight.shape[0]
  nb = PACK_C // PACK_BN
  n2 = n - PACK_C
  grid = (n2 + PACK_BN - 1) // PACK_BN
  return pl.pallas_call(
      _fuse_block,
      grid=(grid,),
      in_specs=[
          pl.BlockSpec((EMB_DIM, PACK_BN), lambda i: (0, i)),
          pl.BlockSpec((EMB_DIM, PACK_BN), lambda i: (0, i + nb)),
          pl.BlockSpec((RANK, PACK_BN), lambda i: (0, i)),
          pl.BlockSpec((RANK, PACK_BN), lambda i: (0, i + nb)),
          pl.BlockSpec((EMB_DIM, RANK), lambda i: (0, 0)),
      ],
      out_specs=pl.BlockSpec((PACK_BN, 2 * EMB_DIM), lambda i: (i, 0)),
      out_shape=jax.ShapeDtypeStruct((n2, 2 * EMB_DIM), jnp.float32),
  )(weight.T, weight.T, lora_A.T, lora_A.T, lora_B.T)


def _make_gather(n_idx):
  assert n_idx % (NW * 2 * CHUNK) == 0
  per_w = n_idx // NW
  n_step = per_w // CHUNK
  mesh = plsc.VectorSubcoreMesh(core_axis_name="c", subcore_axis_name="s")

  @functools.partial(
      pl.kernel,
      out_type=jax.ShapeDtypeStruct((n_idx, EMB_DIM), jnp.float32),
      mesh=mesh,
      scratch_types=[
          pltpu.VMEM((n_step, CHUNK), jnp.int32),
          pltpu.VMEM((2, CHUNK, EMB_DIM), jnp.float32),
          pltpu.SemaphoreType.DMA,
          pltpu.SemaphoreType.DMA,
          pltpu.SemaphoreType.DMA,
          pltpu.SemaphoreType.DMA,
      ],
      compiler_params=pltpu.CompilerParams(use_tc_tiling_on_sc=False),
  )
  def gather_rows(x_hbm, f_hbm, out_hbm, idx_all, rows_v, gs0, gs1, ws0, ws1):
    gs = (gs0, gs1)
    ws = (ws0, ws1)
    wid = lax.axis_index("s") * NUM_CORES + lax.axis_index("c")
    base = wid * per_w
    pltpu.sync_copy(x_hbm.at[wid], idx_all)

    def issue(i, s):
      pltpu.async_copy(f_hbm.at[idx_all.at[i]], rows_v.at[s], gs[s])

    def drain_gather(s):
      pltpu.make_async_copy(f_hbm.at[pl.ds(0, CHUNK)], rows_v.at[s], gs[s]).wait()

    def drain_write(s):
      pltpu.make_async_copy(
          rows_v.at[s], out_hbm.at[pl.ds(0, CHUNK)], ws[s]).wait()

    issue(0, 0)

    def outer(g, carry):
      for b in range(2):
        i = 2 * g + b
        nxt = i + 1

        @pl.when(nxt < n_step)
        def _():
          if b == 1:
            drain_write(0)
          else:

            @pl.when(i >= 1)
            def _():
              drain_write(1)

          issue(nxt, 1 - b)

        drain_gather(b)
        off = base + i * CHUNK
        pltpu.async_copy(rows_v.at[b], out_hbm.at[pl.ds(off, CHUNK)], ws[b])
      return carry

    lax.fori_loop(0, n_step // 2, outer, 0, unroll=False)
    drain_write(0)
    drain_write(1)

  return gather_rows


def kernel(x, weight, lora_A, lora_B):
  b, l = x.shape
  n = b * l
  xi = x.astype(jnp.int32)
  xj = jnp.where(xi < PACK_C, 2 * xi, 2 * (xi - PACK_C) + 1)
  xf = xj.reshape(NW, n // (NW * CHUNK), CHUNK)
  fused2 = _build_fused_table(weight, lora_A, lora_B)
  fused = fused2.reshape(2 * fused2.shape[0], EMB_DIM)
  out = _make_gather(n)(xf, fused)
  return out.reshape(b, l, EMB_DIM)
